# initial kernel scaffold (unmeasured)
import jax
import jax.numpy as jnp
from jax import lax
from jax.experimental import pallas as pl
from jax.experimental.pallas import tpu as pltpu

N_DEV = 32


def kernel(A, B):
    m, k = A.shape
    k2, n = B.shape
    assert k == k2
    chunk = m // N_DEV

    def body(a_ref, b_ref, out_ref, rs_buf, send1, recv1, send2, recv2):
        my = lax.axis_index("i")

        out_ref[...] = jnp.dot(
            a_ref[...], b_ref[...], preferred_element_type=jnp.float32
        )

        p1 = []
        for d in range(1, N_DEV):
            tgt = (my + d) % N_DEV
            rdma = pltpu.make_async_remote_copy(
                src_ref=out_ref.at[pl.ds(tgt * chunk, chunk)],
                dst_ref=rs_buf.at[d],
                send_sem=send1.at[d],
                recv_sem=recv1.at[d],
                device_id=(tgt,),
                device_id_type=pl.DeviceIdType.MESH,
            )
            rdma.start()
            p1.append(rdma)
        for r in p1:
            r.wait_recv()

        rs_buf[0, :, :] = out_ref[pl.ds(my * chunk, chunk), :]
        z = jnp.sum(rs_buf[...], axis=0)
        z = z / (1.0 + jnp.exp(-z))
        out_ref[pl.ds(my * chunk, chunk), :] = z

        for r in p1:
            r.wait_send()

        p2 = []
        for d in range(1, N_DEV):
            tgt = (my + d) % N_DEV
            rdma = pltpu.make_async_remote_copy(
                src_ref=out_ref.at[pl.ds(my * chunk, chunk)],
                dst_ref=out_ref.at[pl.ds(my * chunk, chunk)],
                send_sem=send2.at[d],
                recv_sem=recv2.at[d],
                device_id=(tgt,),
                device_id_type=pl.DeviceIdType.MESH,
            )
            rdma.start()
            p2.append(rdma)
        for r in p2:
            r.wait_recv()
        for r in p2:
            r.wait_send()

    return pl.pallas_call(
        body,
        out_shape=jax.ShapeDtypeStruct((m, n), jnp.float32),
        in_specs=[
            pl.BlockSpec(memory_space=pltpu.VMEM),
            pl.BlockSpec(memory_space=pltpu.VMEM),
        ],
        out_specs=pl.BlockSpec(memory_space=pltpu.VMEM),
        scratch_shapes=[
            pltpu.VMEM((N_DEV, chunk, n), jnp.float32),
            pltpu.SemaphoreType.DMA((N_DEV,)),
            pltpu.SemaphoreType.DMA((N_DEV,)),
            pltpu.SemaphoreType.DMA((N_DEV,)),
            pltpu.SemaphoreType.DMA((N_DEV,)),
        ],
        compiler_params=pltpu.CompilerParams(collective_id=0),
    )(A, B)


# baseline (device time: 273416 ns/iter reference)
import jax
import jax.numpy as jnp
from jax import lax
from jax.experimental import pallas as pl
from jax.experimental.pallas import tpu as pltpu

N_DEV = 32


def kernel(A, B):
    m, k = A.shape
    k2, n = B.shape
    assert k == k2
    chunk = m // N_DEV

    def body(a_ref, b_ref, out_ref, rs_buf, send1, recv1, send2, recv2):
        my = lax.axis_index("i")

        out_ref[...] = jnp.dot(
            a_ref[...], b_ref[...], preferred_element_type=jnp.float32
        )

        p1 = []
        for d in range(1, N_DEV):
            tgt = (my + d) % N_DEV
            rdma = pltpu.make_async_remote_copy(
                src_ref=out_ref.at[pl.ds(tgt * chunk, chunk)],
                dst_ref=rs_buf.at[d],
                send_sem=send1.at[d],
                recv_sem=recv1.at[d],
                device_id=(tgt,),
                device_id_type=pl.DeviceIdType.MESH,
            )
            rdma.start()
            p1.append(rdma)
        for r in p1:
            r.wait_recv()

        rs_buf[0, :, :] = out_ref[pl.ds(my * chunk, chunk), :]
        z = jnp.sum(rs_buf[...], axis=0)
        z = z / (1.0 + jnp.exp(-z))
        out_ref[pl.ds(my * chunk, chunk), :] = z

        for r in p1:
            r.wait_send()

        p2 = []
        for d in range(1, N_DEV):
            tgt = (my + d) % N_DEV
            rdma = pltpu.make_async_remote_copy(
                src_ref=out_ref.at[pl.ds(my * chunk, chunk)],
                dst_ref=out_ref.at[pl.ds(my * chunk, chunk)],
                send_sem=send2.at[d],
                recv_sem=recv2.at[d],
                device_id=(tgt,),
                device_id_type=pl.DeviceIdType.MESH,
            )
            rdma.start()
            p2.append(rdma)
        for r in p2:
            r.wait_recv()
        for r in p2:
            r.wait_send()

    return pl.pallas_call(
        body,
        out_shape=jax.ShapeDtypeStruct((m, n), jnp.float32),
        in_specs=[
            pl.BlockSpec(memory_space=pltpu.VMEM),
            pl.BlockSpec(memory_space=pltpu.VMEM),
        ],
        out_specs=pl.BlockSpec(memory_space=pltpu.VMEM),
        scratch_shapes=[
            pltpu.VMEM((N_DEV, chunk, n), jnp.float32),
            pltpu.SemaphoreType.DMA((N_DEV,)),
            pltpu.SemaphoreType.DMA((N_DEV,)),
            pltpu.SemaphoreType.DMA((N_DEV,)),
            pltpu.SemaphoreType.DMA((N_DEV,)),
        ],
    )(A, B)


# device time: 165371 ns/iter; 1.6533x vs baseline; 1.6533x over previous
import jax
import jax.numpy as jnp
from jax import lax
from jax.experimental import pallas as pl
from jax.experimental.pallas import tpu as pltpu

N_DEV = 32
N_CUBE = 4
N_MEM = 8
N_HALF = 2


def kernel(A, B):
    m_rows, k = A.shape
    k2, n = B.shape
    assert k == k2
    blk = m_rows // N_MEM
    strip = blk // N_CUBE
    nh = n // N_HALF

    def idx(c, mem):
        p = (mem % 4) + 4 * (c % 2)
        zz = (mem // 4) + 2 * (c // 2)
        return 8 * zz + p

    def body(
        a_ref,
        b_ref,
        out_ref,
        rs1_buf,
        acc_ref,
        rs2_buf,
        blk_buf,
        s1_send,
        s1_recv,
        s2_send,
        s2_recv,
        g2_send,
        g2_recv,
        g1_send,
        g1_recv,
    ):
        my = lax.axis_index("i")
        mem = (my % 8) % 4 + 4 * ((my // 8) % 2)
        cube = (my % 8) // 4 + 2 * ((my // 8) // 2)

        mates = [idx(cube, (mem + d) % N_MEM) for d in range(1, N_MEM)]
        partners = [idx((cube + e) % N_CUBE, mem) for e in range(1, N_CUBE)]

        barrier_sem = pltpu.get_barrier_semaphore()
        for nbr in mates + partners:
            pl.semaphore_signal(
                barrier_sem,
                inc=1,
                device_id=(nbr,),
                device_id_type=pl.DeviceIdType.MESH,
            )
        pl.semaphore_wait(barrier_sem, len(mates) + len(partners))

        def stage1(h):
            c0 = h * nh
            descs = []
            for d in range(1, N_MEM):
                mm = (mem + d) % N_MEM
                r0 = mm * blk
                out_ref[pl.ds(r0, blk), pl.ds(c0, nh)] = jnp.dot(
                    a_ref[pl.ds(r0, blk), :],
                    b_ref[:, pl.ds(c0, nh)],
                    preferred_element_type=jnp.float32,
                )
                rdma = pltpu.make_async_remote_copy(
                    src_ref=out_ref.at[pl.ds(r0, blk), pl.ds(c0, nh)],
                    dst_ref=rs1_buf.at[h * N_MEM + d],
                    send_sem=s1_send.at[h * N_MEM + d],
                    recv_sem=s1_recv.at[h * N_MEM + d],
                    device_id=(mates[d - 1],),
                    device_id_type=pl.DeviceIdType.MESH,
                )
                rdma.start()
                descs.append(rdma)
            rs1_buf[h * N_MEM, :, :] = jnp.dot(
                a_ref[pl.ds(mem * blk, blk), :],
                b_ref[:, pl.ds(c0, nh)],
                preferred_element_type=jnp.float32,
            )
            return descs

        p1 = {h: stage1(h) for h in range(N_HALF)}
        p_all = [r for h in range(N_HALF) for r in p1[h]]
        for h in range(N_HALF):
            c0 = h * nh
            acc_ref[h, :, :] = rs1_buf[h * N_MEM, :, :]
            for d in range(1, N_MEM):
                p1[h][d - 1].wait_recv()
                acc_ref[h, :, :] += rs1_buf[h * N_MEM + d, :, :]

            p2 = []
            for e in range(1, N_CUBE):
                cc = (cube + e) % N_CUBE
                rdma = pltpu.make_async_remote_copy(
                    src_ref=acc_ref.at[h, pl.ds(cc * strip, strip)],
                    dst_ref=rs2_buf.at[h * N_CUBE + e],
                    send_sem=s2_send.at[h * N_CUBE + e],
                    recv_sem=s2_recv.at[h * N_CUBE + e],
                    device_id=(partners[e - 1],),
                    device_id_type=pl.DeviceIdType.MESH,
                )
                rdma.start()
                p2.append(rdma)
            rs2_buf[h * N_CUBE, :, :] = acc_ref[h, pl.ds(cube * strip, strip), :]
            for r in p2:
                r.wait_recv()
            z = (
                rs2_buf[h * N_CUBE, :, :]
                + rs2_buf[h * N_CUBE + 1, :, :]
                + rs2_buf[h * N_CUBE + 2, :, :]
                + rs2_buf[h * N_CUBE + 3, :, :]
            )
            z = z / (1.0 + jnp.exp(-z))
            blk_buf[h, pl.ds(cube * strip, strip), :] = z

            p3 = []
            for e in range(1, N_CUBE):
                rdma = pltpu.make_async_remote_copy(
                    src_ref=blk_buf.at[h, pl.ds(cube * strip, strip)],
                    dst_ref=blk_buf.at[h, pl.ds(cube * strip, strip)],
                    send_sem=g2_send.at[h * N_CUBE + e],
                    recv_sem=g2_recv.at[h * N_CUBE + e],
                    device_id=(partners[e - 1],),
                    device_id_type=pl.DeviceIdType.MESH,
                )
                rdma.start()
                p3.append(rdma)
            for r in p3:
                r.wait_recv()

            out_ref[pl.ds(mem * blk, blk), pl.ds(c0, nh)] = blk_buf[h, :, :]
            p4 = []
            for d in range(1, N_MEM):
                rdma = pltpu.make_async_remote_copy(
                    src_ref=blk_buf.at[h],
                    dst_ref=out_ref.at[pl.ds(mem * blk, blk), pl.ds(c0, nh)],
                    send_sem=g1_send.at[h * N_MEM + d],
                    recv_sem=g1_recv.at[h * N_MEM + d],
                    device_id=(mates[d - 1],),
                    device_id_type=pl.DeviceIdType.MESH,
                )
                rdma.start()
                p4.append(rdma)
            p_all = p_all + p2 + p3 + p4
            if h == N_HALF - 1:
                for r in p4:
                    r.wait_recv()
            else:
                p4_prev = p4
        for r in p4_prev:
            r.wait_recv()
        for r in p_all:
            r.wait_send()

    return pl.pallas_call(
        body,
        out_shape=jax.ShapeDtypeStruct((m_rows, n), jnp.float32),
        in_specs=[
            pl.BlockSpec(memory_space=pltpu.VMEM),
            pl.BlockSpec(memory_space=pltpu.VMEM),
        ],
        out_specs=pl.BlockSpec(memory_space=pltpu.VMEM),
        scratch_shapes=[
            pltpu.VMEM((N_HALF * N_MEM, blk, nh), jnp.float32),
            pltpu.VMEM((N_HALF, blk, nh), jnp.float32),
            pltpu.VMEM((N_HALF * N_CUBE, strip, nh), jnp.float32),
            pltpu.VMEM((N_HALF, blk, nh), jnp.float32),
            pltpu.SemaphoreType.DMA((N_HALF * N_MEM,)),
            pltpu.SemaphoreType.DMA((N_HALF * N_MEM,)),
            pltpu.SemaphoreType.DMA((N_HALF * N_CUBE,)),
            pltpu.SemaphoreType.DMA((N_HALF * N_CUBE,)),
            pltpu.SemaphoreType.DMA((N_HALF * N_CUBE,)),
            pltpu.SemaphoreType.DMA((N_HALF * N_CUBE,)),
            pltpu.SemaphoreType.DMA((N_HALF * N_MEM,)),
            pltpu.SemaphoreType.DMA((N_HALF * N_MEM,)),
        ],
        compiler_params=pltpu.CompilerParams(collective_id=0),
    )(A, B)
